# baseline (speedup 1.0000x reference)
import jax
import jax.numpy as jnp
from jax import lax
from jax.experimental import pallas as pl
from jax.experimental.pallas import tpu as pltpu
from jax.experimental.pallas import tpu_sc as plsc

HD2 = 128
CNTW = 16


def _pre_node_body(x_ref, w_ref, b_ref, p0_ref, p1_ref):
    r = jnp.dot(x_ref[...], w_ref[...], preferred_element_type=jnp.float32)
    r = r + b_ref[...]
    p0_ref[...] = r[:, :HD2]
    p1_ref[...] = r[:, HD2:]


def _pre_edge_body(ea_ref, w_ref, q0_ref, q1_ref):
    r = jnp.dot(ea_ref[...], w_ref[...], preferred_element_type=jnp.float32)
    q0_ref[...] = r[:, :HD2]
    q1_ref[...] = r[:, HD2:]


def _post_body(x_ref, s0_ref, s1_ref, cnt_ref, w2a_ref, w2b_ref, b2_ref,
               u1a_ref, u1b_ref, ub1_ref, u2_ref, ub2_ref, o_ref):
    c = cnt_ref[...][:, 0:1]
    summed = (jnp.dot(s0_ref[...], w2a_ref[...], preferred_element_type=jnp.float32)
              + jnp.dot(s1_ref[...], w2b_ref[...], preferred_element_type=jnp.float32)
              + c * b2_ref[...])
    agg = summed / (c + 1e-8)
    h = (jnp.dot(x_ref[...], u1a_ref[...], preferred_element_type=jnp.float32)
         + jnp.dot(agg, u1b_ref[...], preferred_element_type=jnp.float32)
         + ub1_ref[...])
    h = jnp.maximum(h, 0.0)
    o_ref[...] = jnp.dot(h, u2_ref[...], preferred_element_type=jnp.float32) + ub2_ref[...]


def _make_sc_main(n_pad, n_edges, block):
    info = plsc.get_sparse_core_info()
    num_subcores = info.num_subcores
    epw = n_edges // num_subcores
    chunks = epw // block
    rpw = n_pad // num_subcores
    mesh = plsc.VectorSubcoreMesh(core_axis_name="c", subcore_axis_name="s")

    ring = 3
    groups = -(-chunks // ring)

    def body(p0_hbm, p1_hbm, q0_hbm, q1_hbm, src_hbm, dst_hbm, zs_hbm,
             s0_out, s1_out, *scr):
        bufs = [tuple(scr[i * 5:(i + 1) * 5]) for i in range(ring)]
        qrows, semQ = scr[ring * 5], scr[ring * 5 + 1]
        s_sh = scr[ring * 5 + 2]
        c = lax.axis_index("c")
        s = lax.axis_index("s")

        @pl.when(s == 0)
        def _():
            pltpu.sync_copy(zs_hbm, s_sh)

        plsc.subcore_barrier()

        base_w = s * epw

        def run(p_hbm, q_hbm):
            def q_start(ci):
                base = base_w + ci * block
                pltpu.async_copy(q_hbm.at[pl.ds(base, block)], qrows, semQ)

            def q_wait(ci):
                base = base_w + ci * block
                pltpu.make_async_copy(
                    q_hbm.at[pl.ds(base, block)], qrows, semQ).wait()

            def prefetch(ci, buf):
                srcv, dstv, rows, semA, semS = buf
                base = base_w + ci * block
                pltpu.sync_copy(src_hbm.at[pl.ds(base, block)], srcv)
                pltpu.sync_copy(dst_hbm.at[pl.ds(base, block)], dstv)
                pltpu.async_copy(p_hbm.at[srcv], rows, semA)

            def drain_scatter(buf):
                srcv, dstv, rows, semA, semS = buf
                pltpu.make_async_copy(rows, s_sh.at[dstv], semS).wait()

            def process(ci, buf):
                srcv, dstv, rows, semA, semS = buf
                pltpu.make_async_copy(p_hbm.at[srcv], rows, semA).wait()
                q_wait(ci)

                def row(r, cc):
                    for rr in range(4):
                        for j in range(HD2 // 16):
                            sl = pl.ds(j * 16, 16)
                            rows[r * 4 + rr, sl] = jnp.maximum(
                                rows[r * 4 + rr, sl] + qrows[r * 4 + rr, sl], 0.0)
                    return cc
                lax.fori_loop(0, block // 4, row, 0)

                pltpu.async_copy(rows, s_sh.at[dstv], semS, add=True)

                @pl.when(ci + 1 < chunks)
                def _():
                    q_start(ci + 1)

            prefetch(0, bufs[0])
            q_start(0)

            def group(g, carry):
                for b in range(ring):
                    ci = ring * g + b
                    nb = (b + 1) % ring
                    ci_next = ci + 1

                    @pl.when(ci_next < chunks)
                    def _():
                        @pl.when(ci_next >= ring)
                        def _():
                            drain_scatter(bufs[nb])
                        prefetch(ci_next, bufs[nb])

                    @pl.when(ci < chunks)
                    def _():
                        process(ci, bufs[b])
                return carry
            lax.fori_loop(0, groups, group, 0)

            for b in range(ring):
                drain_scatter(bufs[b])

        @pl.when(c == 0)
        def _():
            run(p0_hbm, q0_hbm)

        @pl.when(c == 1)
        def _():
            run(p1_hbm, q1_hbm)

        plsc.subcore_barrier()

        sl = pl.ds(s * rpw, rpw)

        @pl.when(c == 0)
        def _():
            pltpu.sync_copy(s_sh.at[sl], s0_out.at[sl])

        @pl.when(c == 1)
        def _():
            pltpu.sync_copy(s_sh.at[sl], s1_out.at[sl])

    f32 = jnp.float32
    return pl.kernel(
        body,
        out_type=[
            jax.ShapeDtypeStruct((n_pad, HD2), f32),
            jax.ShapeDtypeStruct((n_pad, HD2), f32),
        ],
        mesh=mesh,
        scratch_types=[
            pltpu.VMEM((block,), jnp.int32),
            pltpu.VMEM((block,), jnp.int32),
            pltpu.VMEM((block, HD2), f32),
            pltpu.SemaphoreType.DMA,
            pltpu.SemaphoreType.DMA,
        ] * 3 + [
            pltpu.VMEM((block, HD2), f32),
            pltpu.SemaphoreType.DMA,
            pltpu.VMEM_SHARED((n_pad, HD2), f32),
        ],
    )


def _make_sc_count(n_pad, n_edges, block):
    info = plsc.get_sparse_core_info()
    num_subcores = info.num_subcores
    epw = n_edges // num_subcores
    chunks = epw // block
    rpw = n_pad // num_subcores
    mesh = plsc.VectorSubcoreMesh(
        core_axis_name="c", subcore_axis_name="s", num_cores=1)

    def body(dst_hbm, cnt_out, dstv, hist, red, w16, stage_sh):
        s = lax.axis_index("s")

        def z(r, cc):
            hist[pl.ds(r * 16, 16)] = jnp.zeros((16,), jnp.float32)
            return cc
        lax.fori_loop(0, n_pad // 16, z, 0)

        ones16 = jnp.full((16,), 1.0, jnp.float32)

        def chunk(i, carry):
            base = s * epw + i * block
            pltpu.sync_copy(dst_hbm.at[pl.ds(base, block)], dstv)

            def grp(g, cc):
                idx = dstv[pl.ds(g * 16, 16)]
                plsc.addupdate_scatter(hist, [idx], ones16)
                return cc
            lax.fori_loop(0, block // 16, grp, 0)
            return carry
        lax.fori_loop(0, chunks, chunk, 0)

        pltpu.sync_copy(hist, stage_sh.at[s])
        plsc.subcore_barrier()

        sl = pl.ds(s * rpw, rpw)
        pltpu.sync_copy(stage_sh.at[:, sl], red)

        def redchunk(o, cc):
            sl16 = pl.ds(o * 16, 16)
            acc = red[0, sl16]
            for t in range(1, num_subcores):
                acc = acc + red[t, sl16]
            row_idx = o * 16 + lax.iota(jnp.int32, 16)
            col0 = jnp.zeros((16,), jnp.int32)
            plsc.store_scatter(w16, [row_idx, col0], acc)
            return cc
        lax.fori_loop(0, rpw // 16, redchunk, 0)
        pltpu.sync_copy(w16, cnt_out.at[sl])

    f32 = jnp.float32
    return pl.kernel(
        body,
        out_type=[jax.ShapeDtypeStruct((n_pad, CNTW), f32)],
        mesh=mesh,
        compiler_params=pltpu.CompilerParams(needs_layout_passes=False),
        scratch_types=[
            pltpu.VMEM((block,), jnp.int32),
            pltpu.VMEM((n_pad,), f32),
            pltpu.VMEM((num_subcores, rpw), f32),
            pltpu.VMEM((rpw, CNTW), f32),
            pltpu.VMEM_SHARED((num_subcores, n_pad), f32),
        ],
    )


def kernel(x, edge_index, edge_attr, m0W1, m0b1, m0W2, m0b2,
           m1W1, m1b1, m1W2, m1b2, uW1, ub1, uW2, ub2):
    n, nd = x.shape
    e, ed = edge_attr.shape
    f32 = jnp.float32

    w1n = jnp.concatenate([m0W1[:nd], m1W1[:nd]], axis=1)
    w1e = jnp.concatenate([m0W1[nd:], m1W1[nd:]], axis=1)
    b1 = jnp.concatenate([m0b1, m1b1]).reshape(1, 2 * HD2)
    w2a = jnp.concatenate([m0W2[:, :], m1W2[:, :]], axis=0)[:HD2]
    w2b = jnp.concatenate([m0W2[:, :], m1W2[:, :]], axis=0)[HD2:]
    b2 = (m0b2 + m1b2).reshape(1, HD2)
    u1a = uW1[:nd]
    u1b = uW1[nd:]
    ub1r = ub1.reshape(1, HD2)
    ub2r = ub2.reshape(1, nd)

    src = edge_index[0]
    dst = edge_index[1]
    n_pad = ((n + 255) // 256) * 256
    zs = jnp.zeros((n_pad, HD2), f32)

    sc_count = _make_sc_count(n_pad, e, 2000)
    cnt, = sc_count(dst)

    bn = 1000
    grid_n = n // bn
    p0, p1 = pl.pallas_call(
        _pre_node_body,
        grid=(grid_n,),
        in_specs=[
            pl.BlockSpec((bn, nd), lambda i: (i, 0)),
            pl.BlockSpec((nd, 2 * HD2), lambda i: (0, 0)),
            pl.BlockSpec((1, 2 * HD2), lambda i: (0, 0)),
        ],
        out_specs=[
            pl.BlockSpec((bn, HD2), lambda i: (i, 0)),
            pl.BlockSpec((bn, HD2), lambda i: (i, 0)),
        ],
        out_shape=[
            jax.ShapeDtypeStruct((n, HD2), f32),
            jax.ShapeDtypeStruct((n, HD2), f32),
        ],
    )(x, w1n, b1)

    be = 4000
    grid_e = e // be
    q0, q1 = pl.pallas_call(
        _pre_edge_body,
        grid=(grid_e,),
        in_specs=[
            pl.BlockSpec((be, ed), lambda i: (i, 0)),
            pl.BlockSpec((ed, 2 * HD2), lambda i: (0, 0)),
        ],
        out_specs=[
            pl.BlockSpec((be, HD2), lambda i: (i, 0)),
            pl.BlockSpec((be, HD2), lambda i: (i, 0)),
        ],
        out_shape=[
            jax.ShapeDtypeStruct((e, HD2), f32),
            jax.ShapeDtypeStruct((e, HD2), f32),
        ],
    )(edge_attr, w1e)

    sc_main = _make_sc_main(n_pad, e, 80)
    s0, s1 = sc_main(p0, p1, q0, q1, src, dst, zs)

    out = pl.pallas_call(
        _post_body,
        grid=(grid_n,),
        in_specs=[
            pl.BlockSpec((bn, nd), lambda i: (i, 0)),
            pl.BlockSpec((bn, HD2), lambda i: (i, 0)),
            pl.BlockSpec((bn, HD2), lambda i: (i, 0)),
            pl.BlockSpec((bn, CNTW), lambda i: (i, 0)),
            pl.BlockSpec((HD2, HD2), lambda i: (0, 0)),
            pl.BlockSpec((HD2, HD2), lambda i: (0, 0)),
            pl.BlockSpec((1, HD2), lambda i: (0, 0)),
            pl.BlockSpec((nd, HD2), lambda i: (0, 0)),
            pl.BlockSpec((HD2, HD2), lambda i: (0, 0)),
            pl.BlockSpec((1, HD2), lambda i: (0, 0)),
            pl.BlockSpec((HD2, nd), lambda i: (0, 0)),
            pl.BlockSpec((1, nd), lambda i: (0, 0)),
        ],
        out_specs=pl.BlockSpec((bn, nd), lambda i: (i, 0)),
        out_shape=jax.ShapeDtypeStruct((n, nd), f32),
    )(x, s0, s1, cnt, w2a, w2b, b2, u1a, u1b, ub1r, uW2, ub2r)

    return out


# speedup vs baseline: 1.0168x; 1.0168x over previous
import jax
import jax.numpy as jnp
from jax import lax
from jax.experimental import pallas as pl
from jax.experimental.pallas import tpu as pltpu
from jax.experimental.pallas import tpu_sc as plsc

HD2 = 128
CNTW = 16


def _pre_node_body(x_ref, w_ref, b_ref, p0_ref, p1_ref):
    r = jnp.dot(x_ref[...], w_ref[...], preferred_element_type=jnp.float32)
    r = r + b_ref[...]
    p0_ref[...] = r[:, :HD2]
    p1_ref[...] = r[:, HD2:]


def _pre_edge_body(ea_ref, w_ref, q0_ref, q1_ref):
    r = jnp.dot(ea_ref[...], w_ref[...], preferred_element_type=jnp.float32)
    q0_ref[...] = r[:, :HD2]
    q1_ref[...] = r[:, HD2:]


def _post_body(x_ref, s0_ref, s1_ref, cnt_ref, w2a_ref, w2b_ref, b2_ref,
               u1a_ref, u1b_ref, ub1_ref, u2_ref, ub2_ref, o_ref):
    c = cnt_ref[...][:, 0:1]
    summed = (jnp.dot(s0_ref[...], w2a_ref[...], preferred_element_type=jnp.float32)
              + jnp.dot(s1_ref[...], w2b_ref[...], preferred_element_type=jnp.float32)
              + c * b2_ref[...])
    agg = summed / (c + 1e-8)
    h = (jnp.dot(x_ref[...], u1a_ref[...], preferred_element_type=jnp.float32)
         + jnp.dot(agg, u1b_ref[...], preferred_element_type=jnp.float32)
         + ub1_ref[...])
    h = jnp.maximum(h, 0.0)
    o_ref[...] = jnp.dot(h, u2_ref[...], preferred_element_type=jnp.float32) + ub2_ref[...]


def _make_sc_main(n_pad, n_edges, block):
    info = plsc.get_sparse_core_info()
    num_subcores = info.num_subcores
    epw = n_edges // num_subcores
    chunks = epw // block
    rpw = n_pad // num_subcores
    mesh = plsc.VectorSubcoreMesh(core_axis_name="c", subcore_axis_name="s")

    ring = 3
    groups = -(-chunks // ring)

    def body(p0_hbm, p1_hbm, q0_hbm, q1_hbm, src_hbm, dst_hbm, zs_hbm,
             s0_out, s1_out, *scr):
        bufs = [tuple(scr[i * 5:(i + 1) * 5]) for i in range(ring)]
        qrows, semQ = scr[ring * 5], scr[ring * 5 + 1]
        s_sh = scr[ring * 5 + 2]
        c = lax.axis_index("c")
        s = lax.axis_index("s")

        @pl.when(s == 0)
        def _():
            pltpu.sync_copy(zs_hbm, s_sh)

        plsc.subcore_barrier()

        base_w = s * epw

        def run(p_hbm, q_hbm):
            def q_start(ci):
                base = base_w + ci * block
                pltpu.async_copy(q_hbm.at[pl.ds(base, block)], qrows, semQ)

            def q_wait(ci):
                base = base_w + ci * block
                pltpu.make_async_copy(
                    q_hbm.at[pl.ds(base, block)], qrows, semQ).wait()

            def prefetch(ci, buf):
                srcv, dstv, rows, semA, semS = buf
                base = base_w + ci * block
                pltpu.sync_copy(src_hbm.at[pl.ds(base, block)], srcv)
                pltpu.sync_copy(dst_hbm.at[pl.ds(base, block)], dstv)
                pltpu.async_copy(p_hbm.at[srcv], rows, semA)

            def drain_scatter(buf):
                srcv, dstv, rows, semA, semS = buf
                pltpu.make_async_copy(rows, s_sh.at[dstv], semS).wait()

            def process(ci, buf):
                srcv, dstv, rows, semA, semS = buf
                pltpu.make_async_copy(p_hbm.at[srcv], rows, semA).wait()
                q_wait(ci)

                def row(r, cc):
                    for rr in range(4):
                        for j in range(HD2 // 16):
                            sl = pl.ds(j * 16, 16)
                            rows[r * 4 + rr, sl] = jnp.maximum(
                                rows[r * 4 + rr, sl] + qrows[r * 4 + rr, sl], 0.0)
                    return cc
                lax.fori_loop(0, block // 4, row, 0)

                pltpu.async_copy(rows, s_sh.at[dstv], semS, add=True)

                @pl.when(ci + 1 < chunks)
                def _():
                    q_start(ci + 1)

            prefetch(0, bufs[0])
            q_start(0)

            def group(g, carry):
                for b in range(ring):
                    ci = ring * g + b
                    nb = (b + 1) % ring
                    ci_next = ci + 1

                    @pl.when(ci_next < chunks)
                    def _():
                        @pl.when(ci_next >= ring)
                        def _():
                            drain_scatter(bufs[nb])
                        prefetch(ci_next, bufs[nb])

                    @pl.when(ci < chunks)
                    def _():
                        process(ci, bufs[b])
                return carry
            lax.fori_loop(0, groups, group, 0)

            for b in range(ring):
                drain_scatter(bufs[b])

        @pl.when(c == 0)
        def _():
            run(p0_hbm, q0_hbm)

        @pl.when(c == 1)
        def _():
            run(p1_hbm, q1_hbm)

        plsc.subcore_barrier()

        sl = pl.ds(s * rpw, rpw)

        @pl.when(c == 0)
        def _():
            pltpu.sync_copy(s_sh.at[sl], s0_out.at[sl])

        @pl.when(c == 1)
        def _():
            pltpu.sync_copy(s_sh.at[sl], s1_out.at[sl])

    f32 = jnp.float32
    return pl.kernel(
        body,
        out_type=[
            jax.ShapeDtypeStruct((n_pad, HD2), f32),
            jax.ShapeDtypeStruct((n_pad, HD2), f32),
        ],
        mesh=mesh,
        scratch_types=[
            pltpu.VMEM((block,), jnp.int32),
            pltpu.VMEM((block,), jnp.int32),
            pltpu.VMEM((block, HD2), f32),
            pltpu.SemaphoreType.DMA,
            pltpu.SemaphoreType.DMA,
        ] * 3 + [
            pltpu.VMEM((block, HD2), f32),
            pltpu.SemaphoreType.DMA,
            pltpu.VMEM_SHARED((n_pad, HD2), f32),
        ],
    )


def _make_sc_count(n_pad, n_edges, block):
    info = plsc.get_sparse_core_info()
    num_subcores = info.num_subcores
    epw = n_edges // num_subcores
    chunks = epw // block
    rpw = n_pad // num_subcores
    mesh = plsc.VectorSubcoreMesh(
        core_axis_name="c", subcore_axis_name="s", num_cores=1)

    def body(dst_hbm, cnt_out, dstv, hist, red, w16, stage_sh):
        s = lax.axis_index("s")

        def z(r, cc):
            hist[pl.ds(r * 16, 16)] = jnp.zeros((16,), jnp.float32)
            return cc
        lax.fori_loop(0, n_pad // 16, z, 0)

        ones16 = jnp.full((16,), 1.0, jnp.float32)

        def chunk(i, carry):
            base = s * epw + i * block
            pltpu.sync_copy(dst_hbm.at[pl.ds(base, block)], dstv)

            def grp(g, cc):
                idx = dstv[pl.ds(g * 16, 16)]
                plsc.addupdate_scatter(hist, [idx], ones16)
                return cc
            lax.fori_loop(0, block // 16, grp, 0)
            return carry
        lax.fori_loop(0, chunks, chunk, 0)

        pltpu.sync_copy(hist, stage_sh.at[s])
        plsc.subcore_barrier()

        sl = pl.ds(s * rpw, rpw)
        pltpu.sync_copy(stage_sh.at[:, sl], red)

        def redchunk(o, cc):
            sl16 = pl.ds(o * 16, 16)
            acc = red[0, sl16]
            for t in range(1, num_subcores):
                acc = acc + red[t, sl16]
            row_idx = o * 16 + lax.iota(jnp.int32, 16)
            col0 = jnp.zeros((16,), jnp.int32)
            plsc.store_scatter(w16, [row_idx, col0], acc)
            return cc
        lax.fori_loop(0, rpw // 16, redchunk, 0)
        pltpu.sync_copy(w16, cnt_out.at[sl])

    f32 = jnp.float32
    return pl.kernel(
        body,
        out_type=[jax.ShapeDtypeStruct((n_pad, CNTW), f32)],
        mesh=mesh,
        compiler_params=pltpu.CompilerParams(needs_layout_passes=False),
        scratch_types=[
            pltpu.VMEM((block,), jnp.int32),
            pltpu.VMEM((n_pad,), f32),
            pltpu.VMEM((num_subcores, rpw), f32),
            pltpu.VMEM((rpw, CNTW), f32),
            pltpu.VMEM_SHARED((num_subcores, n_pad), f32),
        ],
    )


def kernel(x, edge_index, edge_attr, m0W1, m0b1, m0W2, m0b2,
           m1W1, m1b1, m1W2, m1b2, uW1, ub1, uW2, ub2):
    n, nd = x.shape
    e, ed = edge_attr.shape
    f32 = jnp.float32

    w1n = jnp.concatenate([m0W1[:nd], m1W1[:nd]], axis=1)
    w1e = jnp.concatenate([m0W1[nd:], m1W1[nd:]], axis=1)
    b1 = jnp.concatenate([m0b1, m1b1]).reshape(1, 2 * HD2)
    w2a = jnp.concatenate([m0W2[:, :], m1W2[:, :]], axis=0)[:HD2]
    w2b = jnp.concatenate([m0W2[:, :], m1W2[:, :]], axis=0)[HD2:]
    b2 = (m0b2 + m1b2).reshape(1, HD2)
    u1a = uW1[:nd]
    u1b = uW1[nd:]
    ub1r = ub1.reshape(1, HD2)
    ub2r = ub2.reshape(1, nd)

    src = edge_index[0]
    dst = edge_index[1]
    n_pad = ((n + 255) // 256) * 256
    zs = jnp.zeros((n_pad, HD2), f32)

    sc_count = _make_sc_count(n_pad, e, 2000)
    cnt, = sc_count(dst)

    bn = 2000
    grid_n = n // bn
    p0, p1 = pl.pallas_call(
        _pre_node_body,
        grid=(grid_n,),
        in_specs=[
            pl.BlockSpec((bn, nd), lambda i: (i, 0)),
            pl.BlockSpec((nd, 2 * HD2), lambda i: (0, 0)),
            pl.BlockSpec((1, 2 * HD2), lambda i: (0, 0)),
        ],
        out_specs=[
            pl.BlockSpec((bn, HD2), lambda i: (i, 0)),
            pl.BlockSpec((bn, HD2), lambda i: (i, 0)),
        ],
        out_shape=[
            jax.ShapeDtypeStruct((n, HD2), f32),
            jax.ShapeDtypeStruct((n, HD2), f32),
        ],
    )(x, w1n, b1)

    be = 8000
    grid_e = e // be
    q0, q1 = pl.pallas_call(
        _pre_edge_body,
        grid=(grid_e,),
        in_specs=[
            pl.BlockSpec((be, ed), lambda i: (i, 0)),
            pl.BlockSpec((ed, 2 * HD2), lambda i: (0, 0)),
        ],
        out_specs=[
            pl.BlockSpec((be, HD2), lambda i: (i, 0)),
            pl.BlockSpec((be, HD2), lambda i: (i, 0)),
        ],
        out_shape=[
            jax.ShapeDtypeStruct((e, HD2), f32),
            jax.ShapeDtypeStruct((e, HD2), f32),
        ],
    )(edge_attr, w1e)

    sc_main = _make_sc_main(n_pad, e, 80)
    s0, s1 = sc_main(p0, p1, q0, q1, src, dst, zs)

    out = pl.pallas_call(
        _post_body,
        grid=(grid_n,),
        in_specs=[
            pl.BlockSpec((bn, nd), lambda i: (i, 0)),
            pl.BlockSpec((bn, HD2), lambda i: (i, 0)),
            pl.BlockSpec((bn, HD2), lambda i: (i, 0)),
            pl.BlockSpec((bn, CNTW), lambda i: (i, 0)),
            pl.BlockSpec((HD2, HD2), lambda i: (0, 0)),
            pl.BlockSpec((HD2, HD2), lambda i: (0, 0)),
            pl.BlockSpec((1, HD2), lambda i: (0, 0)),
            pl.BlockSpec((nd, HD2), lambda i: (0, 0)),
            pl.BlockSpec((HD2, HD2), lambda i: (0, 0)),
            pl.BlockSpec((1, HD2), lambda i: (0, 0)),
            pl.BlockSpec((HD2, nd), lambda i: (0, 0)),
            pl.BlockSpec((1, nd), lambda i: (0, 0)),
        ],
        out_specs=pl.BlockSpec((bn, nd), lambda i: (i, 0)),
        out_shape=jax.ShapeDtypeStruct((n, nd), f32),
    )(x, s0, s1, cnt, w2a, w2b, b2, u1a, u1b, ub1r, uW2, ub2r)

    return out


# speedup vs baseline: 1.0217x; 1.0048x over previous
import jax
import jax.numpy as jnp
from jax import lax
from jax.experimental import pallas as pl
from jax.experimental.pallas import tpu as pltpu
from jax.experimental.pallas import tpu_sc as plsc

HD2 = 128
CNTW = 16


def _pre_node_body(x_ref, w_ref, b_ref, p0_ref, p1_ref):
    r = jnp.dot(x_ref[...], w_ref[...], preferred_element_type=jnp.float32)
    r = r + b_ref[...]
    p0_ref[...] = r[:, :HD2]
    p1_ref[...] = r[:, HD2:]


def _pre_edge_body(ea_ref, w_ref, q0_ref, q1_ref):
    r = jnp.dot(ea_ref[...], w_ref[...], preferred_element_type=jnp.float32)
    q0_ref[...] = r[:, :HD2]
    q1_ref[...] = r[:, HD2:]


def _post_body(x_ref, s0_ref, s1_ref, cnt_ref, w2a_ref, w2b_ref, b2_ref,
               u1a_ref, u1b_ref, ub1_ref, u2_ref, ub2_ref, o_ref):
    c = cnt_ref[...][:, 0:1]
    summed = (jnp.dot(s0_ref[...], w2a_ref[...], preferred_element_type=jnp.float32)
              + jnp.dot(s1_ref[...], w2b_ref[...], preferred_element_type=jnp.float32)
              + c * b2_ref[...])
    agg = summed / (c + 1e-8)
    h = (jnp.dot(x_ref[...], u1a_ref[...], preferred_element_type=jnp.float32)
         + jnp.dot(agg, u1b_ref[...], preferred_element_type=jnp.float32)
         + ub1_ref[...])
    h = jnp.maximum(h, 0.0)
    o_ref[...] = jnp.dot(h, u2_ref[...], preferred_element_type=jnp.float32) + ub2_ref[...]


def _make_sc_main(n_pad, n_edges, block):
    info = plsc.get_sparse_core_info()
    num_subcores = info.num_subcores
    epw = n_edges // num_subcores
    chunks = epw // block
    rpw = n_pad // num_subcores
    mesh = plsc.VectorSubcoreMesh(core_axis_name="c", subcore_axis_name="s")

    ring = 3
    groups = -(-chunks // ring)

    def body(p0_hbm, p1_hbm, q0_hbm, q1_hbm, src_hbm, dst_hbm, zs_hbm,
             s0_out, s1_out, *scr):
        bufs = [tuple(scr[i * 5:(i + 1) * 5]) for i in range(ring)]
        qrows, semQ = scr[ring * 5], scr[ring * 5 + 1]
        s_sh = scr[ring * 5 + 2]
        c = lax.axis_index("c")
        s = lax.axis_index("s")

        @pl.when(s == 0)
        def _():
            pltpu.sync_copy(zs_hbm, s_sh)

        plsc.subcore_barrier()

        base_w = s * epw

        def run(p_hbm, q_hbm):
            def q_start(ci):
                base = base_w + ci * block
                pltpu.async_copy(q_hbm.at[pl.ds(base, block)], qrows, semQ)

            def q_wait(ci):
                base = base_w + ci * block
                pltpu.make_async_copy(
                    q_hbm.at[pl.ds(base, block)], qrows, semQ).wait()

            def prefetch(ci, buf):
                srcv, dstv, rows, semA, semS = buf
                base = base_w + ci * block
                pltpu.sync_copy(src_hbm.at[pl.ds(base, block)], srcv)
                pltpu.sync_copy(dst_hbm.at[pl.ds(base, block)], dstv)
                pltpu.async_copy(p_hbm.at[srcv], rows, semA)

            def drain_scatter(buf):
                srcv, dstv, rows, semA, semS = buf
                pltpu.make_async_copy(rows, s_sh.at[dstv], semS).wait()

            def process(ci, buf):
                srcv, dstv, rows, semA, semS = buf
                pltpu.make_async_copy(p_hbm.at[srcv], rows, semA).wait()
                q_wait(ci)

                def row(r, cc):
                    for rr in range(4):
                        for j in range(HD2 // 16):
                            sl = pl.ds(j * 16, 16)
                            rows[r * 4 + rr, sl] = jnp.maximum(
                                rows[r * 4 + rr, sl] + qrows[r * 4 + rr, sl], 0.0)
                    return cc
                lax.fori_loop(0, block // 4, row, 0)

                pltpu.async_copy(rows, s_sh.at[dstv], semS, add=True)

                @pl.when(ci + 1 < chunks)
                def _():
                    q_start(ci + 1)

            prefetch(0, bufs[0])
            q_start(0)

            def group(g, carry):
                for b in range(ring):
                    ci = ring * g + b
                    nb = (b + 1) % ring
                    ci_next = ci + 1

                    @pl.when(ci_next < chunks)
                    def _():
                        @pl.when(ci_next >= ring)
                        def _():
                            drain_scatter(bufs[nb])
                        prefetch(ci_next, bufs[nb])

                    @pl.when(ci < chunks)
                    def _():
                        process(ci, bufs[b])
                return carry
            lax.fori_loop(0, groups, group, 0)

            for b in range(ring):
                drain_scatter(bufs[b])

        @pl.when(c == 0)
        def _():
            run(p0_hbm, q0_hbm)

        @pl.when(c == 1)
        def _():
            run(p1_hbm, q1_hbm)

        plsc.subcore_barrier()

        sl = pl.ds(s * rpw, rpw)

        @pl.when(c == 0)
        def _():
            pltpu.sync_copy(s_sh.at[sl], s0_out.at[sl])

        @pl.when(c == 1)
        def _():
            pltpu.sync_copy(s_sh.at[sl], s1_out.at[sl])

    f32 = jnp.float32
    return pl.kernel(
        body,
        out_type=[
            jax.ShapeDtypeStruct((n_pad, HD2), f32),
            jax.ShapeDtypeStruct((n_pad, HD2), f32),
        ],
        mesh=mesh,
        scratch_types=[
            pltpu.VMEM((block,), jnp.int32),
            pltpu.VMEM((block,), jnp.int32),
            pltpu.VMEM((block, HD2), f32),
            pltpu.SemaphoreType.DMA,
            pltpu.SemaphoreType.DMA,
        ] * 3 + [
            pltpu.VMEM((block, HD2), f32),
            pltpu.SemaphoreType.DMA,
            pltpu.VMEM_SHARED((n_pad, HD2), f32),
        ],
    )


def _make_sc_count(n_pad, n_edges, block):
    info = plsc.get_sparse_core_info()
    num_subcores = info.num_subcores
    epw = n_edges // num_subcores
    chunks = epw // block
    rpw = n_pad // num_subcores
    mesh = plsc.VectorSubcoreMesh(
        core_axis_name="c", subcore_axis_name="s", num_cores=1)

    def body(dst_hbm, cnt_out, dstv, hist, red, w16, stage_sh):
        s = lax.axis_index("s")

        def z(r, cc):
            hist[pl.ds(r * 16, 16)] = jnp.zeros((16,), jnp.float32)
            return cc
        lax.fori_loop(0, n_pad // 16, z, 0)

        ones16 = jnp.full((16,), 1.0, jnp.float32)

        def chunk(i, carry):
            base = s * epw + i * block
            pltpu.sync_copy(dst_hbm.at[pl.ds(base, block)], dstv)

            def grp(g, cc):
                idx = dstv[pl.ds(g * 16, 16)]
                plsc.addupdate_scatter(hist, [idx], ones16)
                return cc
            lax.fori_loop(0, block // 16, grp, 0)
            return carry
        lax.fori_loop(0, chunks, chunk, 0)

        pltpu.sync_copy(hist, stage_sh.at[s])
        plsc.subcore_barrier()

        sl = pl.ds(s * rpw, rpw)
        pltpu.sync_copy(stage_sh.at[:, sl], red)

        def redchunk(o, cc):
            sl16 = pl.ds(o * 16, 16)
            acc = red[0, sl16]
            for t in range(1, num_subcores):
                acc = acc + red[t, sl16]
            row_idx = o * 16 + lax.iota(jnp.int32, 16)
            col0 = jnp.zeros((16,), jnp.int32)
            plsc.store_scatter(w16, [row_idx, col0], acc)
            return cc
        lax.fori_loop(0, rpw // 16, redchunk, 0)
        pltpu.sync_copy(w16, cnt_out.at[sl])

    f32 = jnp.float32
    return pl.kernel(
        body,
        out_type=[jax.ShapeDtypeStruct((n_pad, CNTW), f32)],
        mesh=mesh,
        compiler_params=pltpu.CompilerParams(needs_layout_passes=False),
        scratch_types=[
            pltpu.VMEM((block,), jnp.int32),
            pltpu.VMEM((n_pad,), f32),
            pltpu.VMEM((num_subcores, rpw), f32),
            pltpu.VMEM((rpw, CNTW), f32),
            pltpu.VMEM_SHARED((num_subcores, n_pad), f32),
        ],
    )


def kernel(x, edge_index, edge_attr, m0W1, m0b1, m0W2, m0b2,
           m1W1, m1b1, m1W2, m1b2, uW1, ub1, uW2, ub2):
    n, nd = x.shape
    e, ed = edge_attr.shape
    f32 = jnp.float32

    w1n = jnp.concatenate([m0W1[:nd], m1W1[:nd]], axis=1)
    w1e = jnp.concatenate([m0W1[nd:], m1W1[nd:]], axis=1)
    b1 = jnp.concatenate([m0b1, m1b1]).reshape(1, 2 * HD2)
    w2a = jnp.concatenate([m0W2[:, :], m1W2[:, :]], axis=0)[:HD2]
    w2b = jnp.concatenate([m0W2[:, :], m1W2[:, :]], axis=0)[HD2:]
    b2 = (m0b2 + m1b2).reshape(1, HD2)
    u1a = uW1[:nd]
    u1b = uW1[nd:]
    ub1r = ub1.reshape(1, HD2)
    ub2r = ub2.reshape(1, nd)

    src = edge_index[0]
    dst = edge_index[1]
    n_pad = ((n + 255) // 256) * 256
    zs = jnp.zeros((n_pad, HD2), f32)

    sc_count = _make_sc_count(n_pad, e, 2000)
    cnt, = sc_count(dst)

    bn = 2000
    grid_n = n // bn
    p0, p1 = pl.pallas_call(
        _pre_node_body,
        grid=(grid_n,),
        in_specs=[
            pl.BlockSpec((bn, nd), lambda i: (i, 0)),
            pl.BlockSpec((nd, 2 * HD2), lambda i: (0, 0)),
            pl.BlockSpec((1, 2 * HD2), lambda i: (0, 0)),
        ],
        out_specs=[
            pl.BlockSpec((bn, HD2), lambda i: (i, 0)),
            pl.BlockSpec((bn, HD2), lambda i: (i, 0)),
        ],
        out_shape=[
            jax.ShapeDtypeStruct((n, HD2), f32),
            jax.ShapeDtypeStruct((n, HD2), f32),
        ],
    )(x, w1n, b1)

    be = 16000
    grid_e = e // be
    q0, q1 = pl.pallas_call(
        _pre_edge_body,
        grid=(grid_e,),
        in_specs=[
            pl.BlockSpec((be, ed), lambda i: (i, 0)),
            pl.BlockSpec((ed, 2 * HD2), lambda i: (0, 0)),
        ],
        out_specs=[
            pl.BlockSpec((be, HD2), lambda i: (i, 0)),
            pl.BlockSpec((be, HD2), lambda i: (i, 0)),
        ],
        out_shape=[
            jax.ShapeDtypeStruct((e, HD2), f32),
            jax.ShapeDtypeStruct((e, HD2), f32),
        ],
    )(edge_attr, w1e)

    sc_main = _make_sc_main(n_pad, e, 80)
    s0, s1 = sc_main(p0, p1, q0, q1, src, dst, zs)

    out = pl.pallas_call(
        _post_body,
        grid=(grid_n,),
        in_specs=[
            pl.BlockSpec((bn, nd), lambda i: (i, 0)),
            pl.BlockSpec((bn, HD2), lambda i: (i, 0)),
            pl.BlockSpec((bn, HD2), lambda i: (i, 0)),
            pl.BlockSpec((bn, CNTW), lambda i: (i, 0)),
            pl.BlockSpec((HD2, HD2), lambda i: (0, 0)),
            pl.BlockSpec((HD2, HD2), lambda i: (0, 0)),
            pl.BlockSpec((1, HD2), lambda i: (0, 0)),
            pl.BlockSpec((nd, HD2), lambda i: (0, 0)),
            pl.BlockSpec((HD2, HD2), lambda i: (0, 0)),
            pl.BlockSpec((1, HD2), lambda i: (0, 0)),
            pl.BlockSpec((HD2, nd), lambda i: (0, 0)),
            pl.BlockSpec((1, nd), lambda i: (0, 0)),
        ],
        out_specs=pl.BlockSpec((bn, nd), lambda i: (i, 0)),
        out_shape=jax.ShapeDtypeStruct((n, nd), f32),
    )(x, s0, s1, cnt, w2a, w2b, b2, u1a, u1b, ub1r, uW2, ub2r)

    return out
